# baseline (speedup 1.0000x reference)
import functools

import jax
import jax.numpy as jnp
from jax import lax
from jax.experimental import pallas as pl
from jax.experimental.pallas import tpu as pltpu
from jax.experimental.pallas import tpu_sc as plsc

_B, _L, _D = 4096, 200, 128
_B_TC = 2048
_B_SC = _B - _B_TC
_NC, _NS = 2, 16
_NW = _NC * _NS
_BPW = _B_SC // _NW
_K = 8
_BB = 128


@functools.partial(
    pl.kernel,
    mesh=plsc.VectorSubcoreMesh(core_axis_name="c", subcore_axis_name="s"),
    out_type=jax.ShapeDtypeStruct((_B_SC, _L, _D), jnp.float32),
    scratch_types=[
        pltpu.VMEM((_L, _D), jnp.float32),
        pltpu.SemaphoreType.DMA,
    ],
)
def _sc_broadcast(table_hbm, out_hbm, tab_v, sem):
    wid = lax.axis_index("s") * _NC + lax.axis_index("c")
    base = wid * _BPW
    pltpu.sync_copy(table_hbm.at[pl.ds(0, _L)], tab_v)

    def chunk(j, c):
        row = base + j * _K
        for t in range(_K):
            pltpu.make_async_copy(tab_v, out_hbm.at[row + t], sem).start()
        for t in range(_K):
            pltpu.make_async_copy(tab_v, out_hbm.at[row + t], sem).wait()
        return c

    lax.fori_loop(0, _BPW // _K, chunk, 0)


def _tc_body(tab_ref, out_ref):
    out_ref[...] = jnp.broadcast_to(tab_ref[...][None, :, :], out_ref.shape)


def _tc_broadcast(table):
    return pl.pallas_call(
        _tc_body,
        grid=(_B_TC // _BB,),
        in_specs=[pl.BlockSpec((_L, _D), lambda i: (0, 0))],
        out_specs=pl.BlockSpec((_BB, _L, _D), lambda i: (i, 0, 0)),
        out_shape=jax.ShapeDtypeStruct((_B_TC, _L, _D), table.dtype),
    )(table)


def kernel(sequence, table):
    out_sc = _sc_broadcast(table)
    out_tc = _tc_broadcast(table)
    return jnp.concatenate([out_tc, out_sc], axis=0)


# speedup vs baseline: 2.4890x; 2.4890x over previous
import functools

import jax
import jax.numpy as jnp
from jax import lax
from jax.experimental import pallas as pl
from jax.experimental.pallas import tpu as pltpu
from jax.experimental.pallas import tpu_sc as plsc

_B, _L, _D = 4096, 200, 128
_NC, _NS = 2, 16
_NW = _NC * _NS
_BPW = _B // _NW
_R = 4
_K = 4


@functools.partial(
    pl.kernel,
    mesh=plsc.VectorSubcoreMesh(core_axis_name="c", subcore_axis_name="s"),
    out_type=jax.ShapeDtypeStruct((_B, _L, _D), jnp.float32),
    scratch_types=[
        pltpu.VMEM((_R, _L, _D), jnp.float32),
        pltpu.SemaphoreType.DMA,
    ],
)
def _sc_broadcast(table_hbm, out_hbm, tab_v, sem):
    wid = lax.axis_index("s") * _NC + lax.axis_index("c")
    base = wid * _BPW
    for r in range(_R):
        pltpu.make_async_copy(table_hbm.at[pl.ds(0, _L)], tab_v.at[r], sem).start()
    for r in range(_R):
        pltpu.make_async_copy(table_hbm.at[pl.ds(0, _L)], tab_v.at[r], sem).wait()

    def chunk(j, c):
        row = base + j * (_K * _R)
        for t in range(_K):
            pltpu.make_async_copy(
                tab_v, out_hbm.at[pl.ds(row + t * _R, _R)], sem
            ).start()
        for t in range(_K):
            pltpu.make_async_copy(
                tab_v, out_hbm.at[pl.ds(row + t * _R, _R)], sem
            ).wait()
        return c

    lax.fori_loop(0, _BPW // (_K * _R), chunk, 0)


def kernel(sequence, table):
    return _sc_broadcast(table)


# speedup vs baseline: 2.6741x; 1.0744x over previous
import functools

import jax
import jax.numpy as jnp
from jax import lax
from jax.experimental import pallas as pl
from jax.experimental.pallas import tpu as pltpu
from jax.experimental.pallas import tpu_sc as plsc

_B, _L, _D = 4096, 200, 128
_NC, _NS = 2, 16
_NW = _NC * _NS
_BPW = _B // _NW
_K = 16


@functools.partial(
    pl.kernel,
    mesh=plsc.VectorSubcoreMesh(core_axis_name="c", subcore_axis_name="s"),
    out_type=jax.ShapeDtypeStruct((_B, _L, _D), jnp.float32),
    scratch_types=[
        pltpu.VMEM((_L, _D), jnp.float32),
        pltpu.SemaphoreType.DMA,
    ],
)
def _sc_broadcast(table_hbm, out_hbm, tab_v, sem):
    wid = lax.axis_index("s") * _NC + lax.axis_index("c")
    base = wid * _BPW
    pltpu.sync_copy(table_hbm.at[pl.ds(0, _L)], tab_v)

    def chunk(j, c):
        row = base + j * _K
        for t in range(_K):
            pltpu.make_async_copy(tab_v, out_hbm.at[row + t], sem).start()
        for t in range(_K):
            pltpu.make_async_copy(tab_v, out_hbm.at[row + t], sem).wait()
        return c

    lax.fori_loop(0, _BPW // _K, chunk, 0)


def kernel(sequence, table):
    return _sc_broadcast(table)


# speedup vs baseline: 2.9894x; 1.1179x over previous
import functools

import jax
import jax.numpy as jnp
from jax import lax
from jax.experimental import pallas as pl
from jax.experimental.pallas import tpu as pltpu

_B, _L, _D = 4096, 200, 128
_BB = 128
_K = 4


def _tc_body(tab_ref, out_ref, buf, sem):
    buf[...] = jnp.broadcast_to(tab_ref[...][None, :, :], (_BB, _L, _D))

    def chunk(j, c):
        row = j * (_K * _BB)
        for t in range(_K):
            pltpu.make_async_copy(
                buf, out_ref.at[pl.ds(row + t * _BB, _BB)], sem
            ).start()
        for t in range(_K):
            pltpu.make_async_copy(
                buf, out_ref.at[pl.ds(row + t * _BB, _BB)], sem
            ).wait()
        return c

    lax.fori_loop(0, _B // (_K * _BB), chunk, 0)


def kernel(sequence, table):
    return pl.pallas_call(
        _tc_body,
        out_specs=pl.BlockSpec(memory_space=pl.ANY),
        out_shape=jax.ShapeDtypeStruct((_B, _L, _D), table.dtype),
        scratch_shapes=[
            pltpu.VMEM((_BB, _L, _D), jnp.float32),
            pltpu.SemaphoreType.DMA,
        ],
    )(table[:_L])
